# baseline (speedup 1.0000x reference)
import functools

import jax
import jax.numpy as jnp
from jax.experimental import pallas as pl
from jax.experimental.pallas import tpu as pltpu

N = 10000
E = 320000
D = 128
K = 16
XP = 16

BE = 2560
BN = 1000


def _silu(v):
    return v * jax.nn.sigmoid(v)


def _edge_body(hi_ref, hj_ref, ee_ref, xi_ref, xj_ref,
               wm1i_ref, wm1j_ref, wm1r_ref, wm1e_ref, bm1_ref,
               wm2_ref, bm2_ref, wx1_ref, bx1_ref, wx2_ref, bx2_ref,
               wp1i_ref, wp1j_ref, wp1r_ref, wp1e_ref, bp1_ref,
               wp2_ref, bp2_ref,
               pm_ref, dx_ref):
    hi = hi_ref[...]
    hj = hj_ref[...]
    ee = ee_ref[...]
    xi = xi_ref[...]
    xj = xj_ref[...]
    rij = xi - xj
    dist_sq = jnp.sum(rij * rij, axis=1, keepdims=True)
    dist = jnp.sqrt(jnp.maximum(dist_sq, 0.0))
    centers = (jax.lax.broadcasted_iota(jnp.int32, (1, K), 1)
               .astype(jnp.float32) * (10.0 / (K - 1)))
    inv_w = jnp.float32(1.0 / (10.0 / K + 1e-12))
    z = (dist - centers) * inv_w
    rbf = jnp.exp(-0.5 * z * z)

    f32 = jnp.float32
    pre_m = (jnp.dot(hi, wm1i_ref[...], preferred_element_type=f32, precision=jax.lax.Precision.HIGHEST)
             + jnp.dot(hj, wm1j_ref[...], preferred_element_type=f32, precision=jax.lax.Precision.HIGHEST)
             + jnp.dot(rbf, wm1r_ref[...], preferred_element_type=f32, precision=jax.lax.Precision.HIGHEST)
             + jnp.dot(ee, wm1e_ref[...], preferred_element_type=f32, precision=jax.lax.Precision.HIGHEST)
             + bm1_ref[...])
    m = _silu(pre_m)
    m = _silu(jnp.dot(m, wm2_ref[...], preferred_element_type=f32, precision=jax.lax.Precision.HIGHEST) + bm2_ref[...])
    g = _silu(jnp.dot(m, wx1_ref[...], preferred_element_type=f32, precision=jax.lax.Precision.HIGHEST) + bx1_ref[...])
    gate = jnp.sum(g * wx2_ref[...], axis=1, keepdims=True) + bx2_ref[0, 0]
    dx_ref[...] = rij * gate

    pre_p = (jnp.dot(hi, wp1i_ref[...], preferred_element_type=f32, precision=jax.lax.Precision.HIGHEST)
             + jnp.dot(hj, wp1j_ref[...], preferred_element_type=f32, precision=jax.lax.Precision.HIGHEST)
             + jnp.dot(rbf, wp1r_ref[...], preferred_element_type=f32, precision=jax.lax.Precision.HIGHEST)
             + jnp.dot(ee, wp1e_ref[...], preferred_element_type=f32, precision=jax.lax.Precision.HIGHEST)
             + bp1_ref[...])
    pm = _silu(pre_p)
    pm = _silu(jnp.dot(pm, wp2_ref[...], preferred_element_type=f32, precision=jax.lax.Precision.HIGHEST) + bp2_ref[...])
    pm_ref[...] = pm


def _edge_stage(hi, hj, ee, xi, xj, Wm1, bm1, Wm2, bm2, Wx1, bx1, Wx2, bx2,
                Wp1, bp1, Wp2, bp2, *, interpret=False):
    e = hi.shape[0]
    be = min(BE, e)
    grid = (e // be,)
    wm1i = Wm1[:D]
    wm1j = Wm1[D:2 * D]
    wm1r = Wm1[2 * D:2 * D + K]
    wm1e = Wm1[2 * D + K:]
    wp1i = Wp1[:D]
    wp1j = Wp1[D:2 * D]
    wp1r = Wp1[2 * D:2 * D + K]
    wp1e = Wp1[2 * D + K:]

    eb = lambda w: pl.BlockSpec((be, w), lambda i: (i, 0))
    full = lambda a: pl.BlockSpec(a.shape, lambda i: (0,) * a.ndim)
    r2 = lambda v: v.reshape(1, -1)

    args = (hi, hj, ee, xi, xj,
            wm1i, wm1j, wm1r, wm1e, r2(bm1),
            Wm2, r2(bm2), Wx1, r2(bx1), r2(Wx2), r2(bx2),
            wp1i, wp1j, wp1r, wp1e, r2(bp1),
            Wp2, r2(bp2))
    in_specs = [eb(D), eb(D), eb(D), eb(XP), eb(XP)] + [full(a) for a in args[5:]]
    pm, dx = pl.pallas_call(
        _edge_body,
        grid=grid,
        in_specs=in_specs,
        out_specs=[eb(D), eb(XP)],
        out_shape=[jax.ShapeDtypeStruct((e, D), jnp.float32),
                   jax.ShapeDtypeStruct((e, XP), jnp.float32)],
        interpret=interpret,
    )(*args)
    return pm, dx


def _node_body(h_ref, xp_ref, pms_ref, dxs_ref,
               wu1a_ref, wu1b_ref, bu1_ref, wu2_ref, bu2_ref,
               gamma_ref, beta_ref,
               hn_ref, xn_ref):
    h = h_ref[...]
    pms = pms_ref[...]
    f32 = jnp.float32
    u = _silu(jnp.dot(h, wu1a_ref[...], preferred_element_type=f32, precision=jax.lax.Precision.HIGHEST)
              + jnp.dot(pms, wu1b_ref[...], preferred_element_type=f32, precision=jax.lax.Precision.HIGHEST)
              + bu1_ref[...])
    upd = jnp.dot(u, wu2_ref[...], preferred_element_type=f32, precision=jax.lax.Precision.HIGHEST) + bu2_ref[...]
    pre = h + upd
    mu = jnp.mean(pre, axis=1, keepdims=True)
    cen = pre - mu
    var = jnp.mean(cen * cen, axis=1, keepdims=True)
    hn_ref[...] = cen * jax.lax.rsqrt(var + 1e-5) * gamma_ref[...] + beta_ref[...]
    xn_ref[...] = xp_ref[...] + dxs_ref[...]


def _node_stage(h, xp, pm_sum, dx_sum, Wu1, bu1, Wu2, bu2, gamma, beta,
                *, interpret=False):
    n = h.shape[0]
    bn = min(BN, n)
    grid = (n // bn,)
    nb = lambda w: pl.BlockSpec((bn, w), lambda i: (i, 0))
    full = lambda a: pl.BlockSpec(a.shape, lambda i: (0,) * a.ndim)
    r2 = lambda v: v.reshape(1, -1)
    args = (h, xp, pm_sum, dx_sum,
            Wu1[:D], Wu1[D:], r2(bu1), Wu2, r2(bu2), r2(gamma), r2(beta))
    in_specs = [nb(D), nb(XP), nb(D), nb(XP)] + [full(a) for a in args[4:]]
    hn, xn = pl.pallas_call(
        _node_body,
        grid=grid,
        in_specs=in_specs,
        out_specs=[nb(D), nb(XP)],
        out_shape=[jax.ShapeDtypeStruct((n, D), jnp.float32),
                   jax.ShapeDtypeStruct((n, XP), jnp.float32)],
        interpret=interpret,
    )(*args)
    return hn, xn


def kernel(h, x, e_emb, Wm1, bm1, Wm2, bm2, Wx1, bx1, Wx2, bx2, Wp1, bp1,
           Wp2, bp2, Wu1, bu1, Wu2, bu2, gamma, beta, edge_index,
           *, interpret=False):
    n = h.shape[0]
    src = edge_index[0]
    dst = edge_index[1]
    xp = jnp.pad(x, ((0, 0), (0, XP - x.shape[1])))

    hi = jnp.take(h, dst, axis=0)
    hj = jnp.take(h, src, axis=0)
    xi = jnp.take(xp, dst, axis=0)
    xj = jnp.take(xp, src, axis=0)

    pm, dx = _edge_stage(hi, hj, e_emb, xi, xj, Wm1, bm1, Wm2, bm2,
                         Wx1, bx1, Wx2, bx2, Wp1, bp1, Wp2, bp2,
                         interpret=interpret)

    pm_sum = jax.ops.segment_sum(pm, dst, num_segments=n)
    dx_sum = jax.ops.segment_sum(dx, dst, num_segments=n)

    hn, xn = _node_stage(h, xp, pm_sum, dx_sum, Wu1, bu1, Wu2, bu2,
                         gamma, beta, interpret=interpret)
    return hn, xn[:, :x.shape[1]]


# speedup vs baseline: 1.2074x; 1.2074x over previous
import functools

import jax
import jax.numpy as jnp
from jax import lax
from jax.experimental import pallas as pl
from jax.experimental.pallas import tpu as pltpu
from jax.experimental.pallas import tpu_sc as plsc

N = 10000
E = 320000
D = 128
K = 16
XP = 16

BE = 2560
BN = 1000

NC = 2
NS = 16
CS = 128
NCHUNK = E // CS
CPT = NCHUNK // NS
NEXTRA = NCHUNK - CPT * NS
IDXROWS = CPT + 4
ROWS_A = 632
ROWS_B = N - 15 * ROWS_A

HIGHEST = jax.lax.Precision.HIGHEST


def _silu(v):
    return v * jax.nn.sigmoid(v)


def _edge_body(hi_ref, hj_ref, ee_ref, xi_ref, xj_ref,
               wm1i_ref, wm1j_ref, wm1r_ref, wm1e_ref, bm1_ref,
               wm2_ref, bm2_ref, wx1_ref, bx1_ref, wx2_ref, bx2_ref,
               wp1i_ref, wp1j_ref, wp1r_ref, wp1e_ref, bp1_ref,
               wp2_ref, bp2_ref,
               pm_ref, dxw_ref):
    hi = hi_ref[...]
    hj = hj_ref[...]
    ee = ee_ref[...]
    xi = xi_ref[...]
    xj = xj_ref[...]
    rij = xi - xj
    dist_sq = jnp.sum(rij * rij, axis=1, keepdims=True)
    dist = jnp.sqrt(jnp.maximum(dist_sq, 0.0))
    centers = (jax.lax.broadcasted_iota(jnp.int32, (1, K), 1)
               .astype(jnp.float32) * (10.0 / (K - 1)))
    inv_w = jnp.float32(1.0 / (10.0 / K + 1e-12))
    z = (dist - centers) * inv_w
    rbf = jnp.exp(-0.5 * z * z)

    f32 = jnp.float32
    pre_m = (jnp.dot(hi, wm1i_ref[...], preferred_element_type=f32, precision=HIGHEST)
             + jnp.dot(hj, wm1j_ref[...], preferred_element_type=f32, precision=HIGHEST)
             + jnp.dot(rbf, wm1r_ref[...], preferred_element_type=f32, precision=HIGHEST)
             + jnp.dot(ee, wm1e_ref[...], preferred_element_type=f32, precision=HIGHEST)
             + bm1_ref[...])
    m = _silu(pre_m)
    m = _silu(jnp.dot(m, wm2_ref[...], preferred_element_type=f32, precision=HIGHEST) + bm2_ref[...])
    g = _silu(jnp.dot(m, wx1_ref[...], preferred_element_type=f32, precision=HIGHEST) + bx1_ref[...])
    gate = jnp.sum(g * wx2_ref[...], axis=1, keepdims=True) + bx2_ref[0, 0]

    pre_p = (jnp.dot(hi, wp1i_ref[...], preferred_element_type=f32, precision=HIGHEST)
             + jnp.dot(hj, wp1j_ref[...], preferred_element_type=f32, precision=HIGHEST)
             + jnp.dot(rbf, wp1r_ref[...], preferred_element_type=f32, precision=HIGHEST)
             + jnp.dot(ee, wp1e_ref[...], preferred_element_type=f32, precision=HIGHEST)
             + bp1_ref[...])
    pm = _silu(pre_p)
    pm = _silu(jnp.dot(pm, wp2_ref[...], preferred_element_type=f32, precision=HIGHEST) + bp2_ref[...])
    pm_ref[...] = pm
    dxw_ref[...] = jnp.pad(rij * gate, ((0, 0), (0, D - XP)))


def _edge_stage(hi, hj, ee, xi, xj, Wm1, bm1, Wm2, bm2, Wx1, bx1, Wx2, bx2,
                Wp1, bp1, Wp2, bp2, *, interpret=False):
    e = hi.shape[0]
    be = min(BE, e)
    grid = (pl.cdiv(e, be),)
    wm1i = Wm1[:D]
    wm1j = Wm1[D:2 * D]
    wm1r = Wm1[2 * D:2 * D + K]
    wm1e = Wm1[2 * D + K:]
    wp1i = Wp1[:D]
    wp1j = Wp1[D:2 * D]
    wp1r = Wp1[2 * D:2 * D + K]
    wp1e = Wp1[2 * D + K:]

    eb = lambda w: pl.BlockSpec((be, w), lambda i: (i, 0))
    full = lambda a: pl.BlockSpec(a.shape, lambda i: (0,) * a.ndim)
    r2 = lambda v: v.reshape(1, -1)

    args = (hi, hj, ee, xi, xj,
            wm1i, wm1j, wm1r, wm1e, r2(bm1),
            Wm2, r2(bm2), Wx1, r2(bx1), r2(Wx2), r2(bx2),
            wp1i, wp1j, wp1r, wp1e, r2(bp1),
            Wp2, r2(bp2))
    in_specs = [eb(D), eb(D), eb(D), eb(XP), eb(XP)] + [full(a) for a in args[5:]]
    pm, dxw = pl.pallas_call(
        _edge_body,
        grid=grid,
        in_specs=in_specs,
        out_specs=[eb(D), eb(D)],
        out_shape=[jax.ShapeDtypeStruct((e, D), jnp.float32),
                   jax.ShapeDtypeStruct((e, D), jnp.float32)],
        interpret=interpret,
    )(*args)
    return pm, dxw


def _sc_scatter_body(pm_hbm, dxw_hbm, idx3_hbm, zero_hbm,
                     pmout_hbm, dxout_hbm,
                     acc_sh, idx_v, row_v):
    c = lax.axis_index("c")
    s = lax.axis_index("s")

    r0 = s * ROWS_A

    @pl.when(s < 15)
    def _():
        pltpu.sync_copy(zero_hbm.at[pl.ds(r0, ROWS_A)],
                        acc_sh.at[pl.ds(r0, ROWS_A)])

    @pl.when(s == 15)
    def _():
        pltpu.sync_copy(zero_hbm.at[pl.ds(15 * ROWS_A, ROWS_B)],
                        acc_sh.at[pl.ds(15 * ROWS_A, ROWS_B)])

    ch0 = s * CPT
    pltpu.sync_copy(idx3_hbm.at[s], idx_v)

    plsc.subcore_barrier()

    def _loop(ed_hbm):
        def step(k, carry):
            pltpu.sync_copy(ed_hbm.at[pl.ds((ch0 + k) * CS, CS)], row_v)
            pltpu.sync_copy(row_v, acc_sh.at[idx_v.at[k]], add=True)
            return carry

        lax.fori_loop(0, CPT, step, 0, unroll=False)

        @pl.when(s < NEXTRA)
        def _():
            pltpu.sync_copy(ed_hbm.at[pl.ds((NS * CPT + s) * CS, CS)], row_v)
            pltpu.sync_copy(row_v, acc_sh.at[idx_v.at[CPT]], add=True)

    @pl.when(c == 0)
    def _():
        _loop(pm_hbm)

    @pl.when(c == 1)
    def _():
        _loop(dxw_hbm)

    plsc.subcore_barrier()

    def _writeout(out_hbm):
        @pl.when(s < 15)
        def _():
            pltpu.sync_copy(acc_sh.at[pl.ds(r0, ROWS_A)],
                            out_hbm.at[pl.ds(r0, ROWS_A)])

        @pl.when(s == 15)
        def _():
            pltpu.sync_copy(acc_sh.at[pl.ds(15 * ROWS_A, ROWS_B)],
                            out_hbm.at[pl.ds(15 * ROWS_A, ROWS_B)])

    @pl.when(c == 0)
    def _():
        _writeout(pmout_hbm)

    @pl.when(c == 1)
    def _():
        _writeout(dxout_hbm)


def _scatter_stage(pm, dxw, dst):
    dst2 = dst.reshape(NCHUNK, CS)
    main = dst2[:NS * CPT].reshape(NS, CPT, CS)
    extra = jnp.zeros((NS, IDXROWS - CPT, CS), jnp.int32)
    extra = extra.at[:NEXTRA, 0].set(dst2[NS * CPT:])
    idx3 = jnp.concatenate([main, extra], axis=1)
    zero = jnp.zeros((N, D), jnp.float32)
    mesh = plsc.VectorSubcoreMesh(core_axis_name="c", subcore_axis_name="s")
    pm_sum, dxw_sum = pl.kernel(
        _sc_scatter_body,
        out_type=[jax.ShapeDtypeStruct((N, D), jnp.float32),
                  jax.ShapeDtypeStruct((N, D), jnp.float32)],
        mesh=mesh,
        scratch_types=[
            pltpu.VMEM_SHARED((N, D), jnp.float32),
            pltpu.VMEM((IDXROWS, CS), jnp.int32),
            pltpu.VMEM((CS, D), jnp.float32),
        ],
    )(pm, dxw, idx3, zero)
    return pm_sum, dxw_sum


def _node_body(h_ref, xp_ref, pms_ref, dxs_ref,
               wu1a_ref, wu1b_ref, bu1_ref, wu2_ref, bu2_ref,
               gamma_ref, beta_ref,
               hn_ref, xn_ref):
    h = h_ref[...]
    pms = pms_ref[...]
    f32 = jnp.float32
    u = _silu(jnp.dot(h, wu1a_ref[...], preferred_element_type=f32, precision=HIGHEST)
              + jnp.dot(pms, wu1b_ref[...], preferred_element_type=f32, precision=HIGHEST)
              + bu1_ref[...])
    upd = jnp.dot(u, wu2_ref[...], preferred_element_type=f32, precision=HIGHEST) + bu2_ref[...]
    pre = h + upd
    mu = jnp.mean(pre, axis=1, keepdims=True)
    cen = pre - mu
    var = jnp.mean(cen * cen, axis=1, keepdims=True)
    hn_ref[...] = cen * jax.lax.rsqrt(var + 1e-5) * gamma_ref[...] + beta_ref[...]
    xn_ref[...] = xp_ref[...] + dxs_ref[:, :XP]


def _node_stage(h, xp, pm_sum, dxw_sum, Wu1, bu1, Wu2, bu2, gamma, beta,
                *, interpret=False):
    n = h.shape[0]
    bn = min(BN, n)
    grid = (pl.cdiv(n, bn),)
    nb = lambda w: pl.BlockSpec((bn, w), lambda i: (i, 0))
    full = lambda a: pl.BlockSpec(a.shape, lambda i: (0,) * a.ndim)
    r2 = lambda v: v.reshape(1, -1)
    args = (h, xp, pm_sum, dxw_sum,
            Wu1[:D], Wu1[D:], r2(bu1), Wu2, r2(bu2), r2(gamma), r2(beta))
    in_specs = [nb(D), nb(XP), nb(D), nb(D)] + [full(a) for a in args[4:]]
    hn, xn = pl.pallas_call(
        _node_body,
        grid=grid,
        in_specs=in_specs,
        out_specs=[nb(D), nb(XP)],
        out_shape=[jax.ShapeDtypeStruct((n, D), jnp.float32),
                   jax.ShapeDtypeStruct((n, XP), jnp.float32)],
        interpret=interpret,
    )(*args)
    return hn, xn


def kernel(h, x, e_emb, Wm1, bm1, Wm2, bm2, Wx1, bx1, Wx2, bx2, Wp1, bp1,
           Wp2, bp2, Wu1, bu1, Wu2, bu2, gamma, beta, edge_index,
           *, interpret=False):
    n = h.shape[0]
    src = edge_index[0]
    dst = edge_index[1]
    xp = jnp.pad(x, ((0, 0), (0, XP - x.shape[1])))

    hi = jnp.take(h, dst, axis=0)
    hj = jnp.take(h, src, axis=0)
    xi = jnp.take(xp, dst, axis=0)
    xj = jnp.take(xp, src, axis=0)

    pm, dxw = _edge_stage(hi, hj, e_emb, xi, xj, Wm1, bm1, Wm2, bm2,
                          Wx1, bx1, Wx2, bx2, Wp1, bp1, Wp2, bp2,
                          interpret=interpret)

    if interpret:
        pm_sum = jax.ops.segment_sum(pm, dst, num_segments=n)
        dxw_sum = jax.ops.segment_sum(dxw, dst, num_segments=n)
    else:
        pm_sum, dxw_sum = _scatter_stage(pm, dxw, dst)

    hn, xn = _node_stage(h, xp, pm_sum, dxw_sum, Wu1, bu1, Wu2, bu2,
                         gamma, beta, interpret=interpret)
    return hn, xn[:, :x.shape[1]]


# speedup vs baseline: 1.8589x; 1.5396x over previous
import functools

import jax
import jax.numpy as jnp
from jax import lax
from jax.experimental import pallas as pl
from jax.experimental.pallas import tpu as pltpu
from jax.experimental.pallas import tpu_sc as plsc

N = 10000
E = 320000
D = 128
K = 16
XP = 16

BE = 2560
BN = 1000

NC = 2
NS = 16
CS = 128
NCHUNK = E // CS
CPT = NCHUNK // NS
NEXTRA = NCHUNK - CPT * NS
IDXROWS = CPT + 4
ROWS_A = 632
ROWS_B = N - 15 * ROWS_A


def _silu(v):
    return v * jax.nn.sigmoid(v)


def _edge_body(hi_ref, hj_ref, ee_ref, xi_ref, xj_ref,
               wm1i_ref, wm1j_ref, wm1r_ref, wm1e_ref, bm1_ref,
               wm2_ref, bm2_ref, wx1_ref, bx1_ref, wx2_ref, bx2_ref,
               wp1i_ref, wp1j_ref, wp1r_ref, wp1e_ref, bp1_ref,
               wp2_ref, bp2_ref,
               pm_ref, dxw_ref):
    hi = hi_ref[...]
    hj = hj_ref[...]
    ee = ee_ref[...]
    xi = xi_ref[...]
    xj = xj_ref[...]
    rij = xi - xj
    dist_sq = jnp.sum(rij * rij, axis=1, keepdims=True)
    dist = jnp.sqrt(jnp.maximum(dist_sq, 0.0))
    centers = (jax.lax.broadcasted_iota(jnp.int32, (1, K), 1)
               .astype(jnp.float32) * (10.0 / (K - 1)))
    inv_w = jnp.float32(1.0 / (10.0 / K + 1e-12))
    z = (dist - centers) * inv_w
    rbf = jnp.exp(-0.5 * z * z)

    f32 = jnp.float32
    pre_m = (jnp.dot(hi, wm1i_ref[...], preferred_element_type=f32)
             + jnp.dot(hj, wm1j_ref[...], preferred_element_type=f32)
             + jnp.dot(rbf, wm1r_ref[...], preferred_element_type=f32)
             + jnp.dot(ee, wm1e_ref[...], preferred_element_type=f32)
             + bm1_ref[...])
    m = _silu(pre_m)
    m = _silu(jnp.dot(m, wm2_ref[...], preferred_element_type=f32) + bm2_ref[...])
    g = _silu(jnp.dot(m, wx1_ref[...], preferred_element_type=f32) + bx1_ref[...])
    gate = jnp.sum(g * wx2_ref[...], axis=1, keepdims=True) + bx2_ref[0, 0]

    pre_p = (jnp.dot(hi, wp1i_ref[...], preferred_element_type=f32)
             + jnp.dot(hj, wp1j_ref[...], preferred_element_type=f32)
             + jnp.dot(rbf, wp1r_ref[...], preferred_element_type=f32)
             + jnp.dot(ee, wp1e_ref[...], preferred_element_type=f32)
             + bp1_ref[...])
    pm = _silu(pre_p)
    pm = _silu(jnp.dot(pm, wp2_ref[...], preferred_element_type=f32) + bp2_ref[...])
    pm_ref[...] = pm
    dxw_ref[...] = jnp.pad(rij * gate, ((0, 0), (0, D - XP)))


def _edge_stage(hi, hj, ee, xi, xj, Wm1, bm1, Wm2, bm2, Wx1, bx1, Wx2, bx2,
                Wp1, bp1, Wp2, bp2, *, interpret=False):
    e = hi.shape[0]
    be = min(BE, e)
    grid = (pl.cdiv(e, be),)
    wm1i = Wm1[:D]
    wm1j = Wm1[D:2 * D]
    wm1r = Wm1[2 * D:2 * D + K]
    wm1e = Wm1[2 * D + K:]
    wp1i = Wp1[:D]
    wp1j = Wp1[D:2 * D]
    wp1r = Wp1[2 * D:2 * D + K]
    wp1e = Wp1[2 * D + K:]

    eb = lambda w: pl.BlockSpec((be, w), lambda i: (i, 0))
    full = lambda a: pl.BlockSpec(a.shape, lambda i: (0,) * a.ndim)
    r2 = lambda v: v.reshape(1, -1)

    args = (hi, hj, ee, xi, xj,
            wm1i, wm1j, wm1r, wm1e, r2(bm1),
            Wm2, r2(bm2), Wx1, r2(bx1), r2(Wx2), r2(bx2),
            wp1i, wp1j, wp1r, wp1e, r2(bp1),
            Wp2, r2(bp2))
    in_specs = [eb(D), eb(D), eb(D), eb(XP), eb(XP)] + [full(a) for a in args[5:]]
    pm, dxw = pl.pallas_call(
        _edge_body,
        grid=grid,
        in_specs=in_specs,
        out_specs=[eb(D), eb(D)],
        out_shape=[jax.ShapeDtypeStruct((e, D), jnp.float32),
                   jax.ShapeDtypeStruct((e, D), jnp.float32)],
        interpret=interpret,
    )(*args)
    return pm, dxw


def _sc_scatter_body(pm_hbm, dxw_hbm, idx3_hbm, zero_hbm,
                     pmout_hbm, dxout_hbm,
                     acc_sh, idx_v, row_v):
    c = lax.axis_index("c")
    s = lax.axis_index("s")

    r0 = s * ROWS_A

    @pl.when(s < 15)
    def _():
        pltpu.sync_copy(zero_hbm.at[pl.ds(r0, ROWS_A)],
                        acc_sh.at[pl.ds(r0, ROWS_A)])

    @pl.when(s == 15)
    def _():
        pltpu.sync_copy(zero_hbm.at[pl.ds(15 * ROWS_A, ROWS_B)],
                        acc_sh.at[pl.ds(15 * ROWS_A, ROWS_B)])

    ch0 = s * CPT
    pltpu.sync_copy(idx3_hbm.at[s], idx_v)

    plsc.subcore_barrier()

    def _loop(ed_hbm):
        def step(k, carry):
            pltpu.sync_copy(ed_hbm.at[pl.ds((ch0 + k) * CS, CS)], row_v)
            pltpu.sync_copy(row_v, acc_sh.at[idx_v.at[k]], add=True)
            return carry

        lax.fori_loop(0, CPT, step, 0, unroll=False)

        @pl.when(s < NEXTRA)
        def _():
            pltpu.sync_copy(ed_hbm.at[pl.ds((NS * CPT + s) * CS, CS)], row_v)
            pltpu.sync_copy(row_v, acc_sh.at[idx_v.at[CPT]], add=True)

    @pl.when(c == 0)
    def _():
        _loop(pm_hbm)

    @pl.when(c == 1)
    def _():
        _loop(dxw_hbm)

    plsc.subcore_barrier()

    def _writeout(out_hbm):
        @pl.when(s < 15)
        def _():
            pltpu.sync_copy(acc_sh.at[pl.ds(r0, ROWS_A)],
                            out_hbm.at[pl.ds(r0, ROWS_A)])

        @pl.when(s == 15)
        def _():
            pltpu.sync_copy(acc_sh.at[pl.ds(15 * ROWS_A, ROWS_B)],
                            out_hbm.at[pl.ds(15 * ROWS_A, ROWS_B)])

    @pl.when(c == 0)
    def _():
        _writeout(pmout_hbm)

    @pl.when(c == 1)
    def _():
        _writeout(dxout_hbm)


def _scatter_stage(pm, dxw, dst):
    dst2 = dst.reshape(NCHUNK, CS)
    main = dst2[:NS * CPT].reshape(NS, CPT, CS)
    extra = jnp.zeros((NS, IDXROWS - CPT, CS), jnp.int32)
    extra = extra.at[:NEXTRA, 0].set(dst2[NS * CPT:])
    idx3 = jnp.concatenate([main, extra], axis=1)
    zero = jnp.zeros((N, D), jnp.float32)
    mesh = plsc.VectorSubcoreMesh(core_axis_name="c", subcore_axis_name="s")
    pm_sum, dxw_sum = pl.kernel(
        _sc_scatter_body,
        out_type=[jax.ShapeDtypeStruct((N, D), jnp.float32),
                  jax.ShapeDtypeStruct((N, D), jnp.float32)],
        mesh=mesh,
        scratch_types=[
            pltpu.VMEM_SHARED((N, D), jnp.float32),
            pltpu.VMEM((IDXROWS, CS), jnp.int32),
            pltpu.VMEM((CS, D), jnp.float32),
        ],
    )(pm, dxw, idx3, zero)
    return pm_sum, dxw_sum


def _node_body(h_ref, xp_ref, pms_ref, dxs_ref,
               wu1a_ref, wu1b_ref, bu1_ref, wu2_ref, bu2_ref,
               gamma_ref, beta_ref,
               hn_ref, xn_ref):
    h = h_ref[...]
    pms = pms_ref[...]
    f32 = jnp.float32
    u = _silu(jnp.dot(h, wu1a_ref[...], preferred_element_type=f32)
              + jnp.dot(pms, wu1b_ref[...], preferred_element_type=f32)
              + bu1_ref[...])
    upd = jnp.dot(u, wu2_ref[...], preferred_element_type=f32) + bu2_ref[...]
    pre = h + upd
    mu = jnp.mean(pre, axis=1, keepdims=True)
    cen = pre - mu
    var = jnp.mean(cen * cen, axis=1, keepdims=True)
    hn_ref[...] = cen * jax.lax.rsqrt(var + 1e-5) * gamma_ref[...] + beta_ref[...]
    xn_ref[...] = xp_ref[...] + dxs_ref[:, :XP]


def _node_stage(h, xp, pm_sum, dxw_sum, Wu1, bu1, Wu2, bu2, gamma, beta,
                *, interpret=False):
    n = h.shape[0]
    bn = min(BN, n)
    grid = (pl.cdiv(n, bn),)
    nb = lambda w: pl.BlockSpec((bn, w), lambda i: (i, 0))
    full = lambda a: pl.BlockSpec(a.shape, lambda i: (0,) * a.ndim)
    r2 = lambda v: v.reshape(1, -1)
    args = (h, xp, pm_sum, dxw_sum,
            Wu1[:D], Wu1[D:], r2(bu1), Wu2, r2(bu2), r2(gamma), r2(beta))
    in_specs = [nb(D), nb(XP), nb(D), nb(D)] + [full(a) for a in args[4:]]
    hn, xn = pl.pallas_call(
        _node_body,
        grid=grid,
        in_specs=in_specs,
        out_specs=[nb(D), nb(XP)],
        out_shape=[jax.ShapeDtypeStruct((n, D), jnp.float32),
                   jax.ShapeDtypeStruct((n, XP), jnp.float32)],
        interpret=interpret,
    )(*args)
    return hn, xn


def kernel(h, x, e_emb, Wm1, bm1, Wm2, bm2, Wx1, bx1, Wx2, bx2, Wp1, bp1,
           Wp2, bp2, Wu1, bu1, Wu2, bu2, gamma, beta, edge_index,
           *, interpret=False):
    n = h.shape[0]
    src = edge_index[0]
    dst = edge_index[1]
    xp = jnp.pad(x, ((0, 0), (0, XP - x.shape[1])))

    hi = jnp.take(h, dst, axis=0)
    hj = jnp.take(h, src, axis=0)
    xi = jnp.take(xp, dst, axis=0)
    xj = jnp.take(xp, src, axis=0)

    pm, dxw = _edge_stage(hi, hj, e_emb, xi, xj, Wm1, bm1, Wm2, bm2,
                          Wx1, bx1, Wx2, bx2, Wp1, bp1, Wp2, bp2,
                          interpret=interpret)

    if interpret:
        pm_sum = jax.ops.segment_sum(pm, dst, num_segments=n)
        dxw_sum = jax.ops.segment_sum(dxw, dst, num_segments=n)
    else:
        pm_sum, dxw_sum = _scatter_stage(pm, dxw, dst)

    hn, xn = _node_stage(h, xp, pm_sum, dxw_sum, Wu1, bu1, Wu2, bu2,
                         gamma, beta, interpret=interpret)
    return hn, xn[:, :x.shape[1]]


# speedup vs baseline: 2.9221x; 1.5720x over previous
import functools

import jax
import jax.numpy as jnp
from jax import lax
from jax.experimental import pallas as pl
from jax.experimental.pallas import tpu as pltpu
from jax.experimental.pallas import tpu_sc as plsc

N = 10000
E = 320000
D = 128
K = 16
XP = 16

BE = 2560
BN = 1000

NC = 2
NS = 16
CS = 128
NCHUNK = E // CS
CPT = NCHUNK // NS
NEXTRA = NCHUNK - CPT * NS
IDXROWS = CPT + 4
ROWS_A = 632
ROWS_B = N - 15 * ROWS_A


def _silu(v):
    return v * jax.nn.sigmoid(v)


def _sc_gather_body(h_hbm, dsti_hbm, srci_hbm,
                    hi_hbm, hj_hbm,
                    idx_v, rowh_v, semh):
    c = lax.axis_index("c")
    s = lax.axis_index("s")
    ch0 = s * CPT

    def _run(idx3_hbm, hout_hbm):
        pltpu.sync_copy(idx3_hbm.at[s], idx_v)

        def step(k, carry):
            idx = idx_v.at[k]
            pltpu.async_copy(h_hbm.at[idx], rowh_v, semh).wait()
            pltpu.sync_copy(rowh_v, hout_hbm.at[pl.ds((ch0 + k) * CS, CS)])
            return carry

        lax.fori_loop(0, CPT, step, 0, unroll=False)

        @pl.when(s < NEXTRA)
        def _():
            idx = idx_v.at[CPT]
            o = (NS * CPT + s) * CS
            pltpu.async_copy(h_hbm.at[idx], rowh_v, semh).wait()
            pltpu.sync_copy(rowh_v, hout_hbm.at[pl.ds(o, CS)])

    @pl.when(c == 0)
    def _():
        _run(dsti_hbm, hi_hbm)

    @pl.when(c == 1)
    def _():
        _run(srci_hbm, hj_hbm)


def _gather_stage(h, dst3, src3):
    mesh = plsc.VectorSubcoreMesh(core_axis_name="c", subcore_axis_name="s")
    hi, hj = pl.kernel(
        _sc_gather_body,
        out_type=[jax.ShapeDtypeStruct((E, D), jnp.float32),
                  jax.ShapeDtypeStruct((E, D), jnp.float32)],
        mesh=mesh,
        scratch_types=[
            pltpu.VMEM((IDXROWS, CS), jnp.int32),
            pltpu.VMEM((CS, D), jnp.float32),
            pltpu.SemaphoreType.DMA,
        ],
    )(h, dst3, src3)
    return hi, hj


def _edge_body(hi_ref, hj_ref, ee_ref, xi_ref, xj_ref,
               wm1i_ref, wm1j_ref, wm1r_ref, wm1e_ref, bm1_ref,
               wm2_ref, bm2_ref, wx1_ref, bx1_ref, wx2_ref, bx2_ref,
               wp1i_ref, wp1j_ref, wp1r_ref, wp1e_ref, bp1_ref,
               wp2_ref, bp2_ref,
               pm_ref, dxw_ref):
    hi = hi_ref[...]
    hj = hj_ref[...]
    ee = ee_ref[...]
    xi = xi_ref[...]
    xj = xj_ref[...]
    rij = xi - xj
    dist_sq = jnp.sum(rij * rij, axis=1, keepdims=True)
    dist = jnp.sqrt(jnp.maximum(dist_sq, 0.0))
    centers = (jax.lax.broadcasted_iota(jnp.int32, (1, K), 1)
               .astype(jnp.float32) * (10.0 / (K - 1)))
    inv_w = jnp.float32(1.0 / (10.0 / K + 1e-12))
    z = (dist - centers) * inv_w
    rbf = jnp.exp(-0.5 * z * z)

    f32 = jnp.float32
    pre_m = (jnp.dot(hi, wm1i_ref[...], preferred_element_type=f32)
             + jnp.dot(hj, wm1j_ref[...], preferred_element_type=f32)
             + jnp.dot(rbf, wm1r_ref[...], preferred_element_type=f32)
             + jnp.dot(ee, wm1e_ref[...], preferred_element_type=f32)
             + bm1_ref[...])
    m = _silu(pre_m)
    m = _silu(jnp.dot(m, wm2_ref[...], preferred_element_type=f32) + bm2_ref[...])
    g = _silu(jnp.dot(m, wx1_ref[...], preferred_element_type=f32) + bx1_ref[...])
    gate = jnp.sum(g * wx2_ref[...], axis=1, keepdims=True) + bx2_ref[0, 0]

    pre_p = (jnp.dot(hi, wp1i_ref[...], preferred_element_type=f32)
             + jnp.dot(hj, wp1j_ref[...], preferred_element_type=f32)
             + jnp.dot(rbf, wp1r_ref[...], preferred_element_type=f32)
             + jnp.dot(ee, wp1e_ref[...], preferred_element_type=f32)
             + bp1_ref[...])
    pm = _silu(pre_p)
    pm = _silu(jnp.dot(pm, wp2_ref[...], preferred_element_type=f32) + bp2_ref[...])
    pm_ref[...] = pm
    dxw_ref[...] = jnp.pad(rij * gate, ((0, 0), (0, D - XP)))


def _edge_stage(hi, hj, ee, xi, xj, Wm1, bm1, Wm2, bm2, Wx1, bx1, Wx2, bx2,
                Wp1, bp1, Wp2, bp2, *, interpret=False):
    e = hi.shape[0]
    be = min(BE, e)
    grid = (pl.cdiv(e, be),)
    wm1i = Wm1[:D]
    wm1j = Wm1[D:2 * D]
    wm1r = Wm1[2 * D:2 * D + K]
    wm1e = Wm1[2 * D + K:]
    wp1i = Wp1[:D]
    wp1j = Wp1[D:2 * D]
    wp1r = Wp1[2 * D:2 * D + K]
    wp1e = Wp1[2 * D + K:]

    eb = lambda w: pl.BlockSpec((be, w), lambda i: (i, 0))
    full = lambda a: pl.BlockSpec(a.shape, lambda i: (0,) * a.ndim)
    r2 = lambda v: v.reshape(1, -1)

    args = (hi, hj, ee, xi, xj,
            wm1i, wm1j, wm1r, wm1e, r2(bm1),
            Wm2, r2(bm2), Wx1, r2(bx1), r2(Wx2), r2(bx2),
            wp1i, wp1j, wp1r, wp1e, r2(bp1),
            Wp2, r2(bp2))
    in_specs = [eb(D), eb(D), eb(D), eb(XP), eb(XP)] + [full(a) for a in args[5:]]
    pm, dxw = pl.pallas_call(
        _edge_body,
        grid=grid,
        in_specs=in_specs,
        out_specs=[eb(D), eb(D)],
        out_shape=[jax.ShapeDtypeStruct((e, D), jnp.float32),
                   jax.ShapeDtypeStruct((e, D), jnp.float32)],
        interpret=interpret,
    )(*args)
    return pm, dxw


def _sc_scatter_body(pm_hbm, dxw_hbm, idx3_hbm, zero_hbm,
                     pmout_hbm, dxout_hbm,
                     acc_sh, idx_v, row_v):
    c = lax.axis_index("c")
    s = lax.axis_index("s")

    r0 = s * ROWS_A

    @pl.when(s < 15)
    def _():
        pltpu.sync_copy(zero_hbm.at[pl.ds(r0, ROWS_A)],
                        acc_sh.at[pl.ds(r0, ROWS_A)])

    @pl.when(s == 15)
    def _():
        pltpu.sync_copy(zero_hbm.at[pl.ds(15 * ROWS_A, ROWS_B)],
                        acc_sh.at[pl.ds(15 * ROWS_A, ROWS_B)])

    ch0 = s * CPT
    pltpu.sync_copy(idx3_hbm.at[s], idx_v)

    plsc.subcore_barrier()

    def _loop(ed_hbm):
        def step(k, carry):
            pltpu.sync_copy(ed_hbm.at[pl.ds((ch0 + k) * CS, CS)], row_v)
            pltpu.sync_copy(row_v, acc_sh.at[idx_v.at[k]], add=True)
            return carry

        lax.fori_loop(0, CPT, step, 0, unroll=False)

        @pl.when(s < NEXTRA)
        def _():
            pltpu.sync_copy(ed_hbm.at[pl.ds((NS * CPT + s) * CS, CS)], row_v)
            pltpu.sync_copy(row_v, acc_sh.at[idx_v.at[CPT]], add=True)

    @pl.when(c == 0)
    def _():
        _loop(pm_hbm)

    @pl.when(c == 1)
    def _():
        _loop(dxw_hbm)

    plsc.subcore_barrier()

    def _writeout(out_hbm):
        @pl.when(s < 15)
        def _():
            pltpu.sync_copy(acc_sh.at[pl.ds(r0, ROWS_A)],
                            out_hbm.at[pl.ds(r0, ROWS_A)])

        @pl.when(s == 15)
        def _():
            pltpu.sync_copy(acc_sh.at[pl.ds(15 * ROWS_A, ROWS_B)],
                            out_hbm.at[pl.ds(15 * ROWS_A, ROWS_B)])

    @pl.when(c == 0)
    def _():
        _writeout(pmout_hbm)

    @pl.when(c == 1)
    def _():
        _writeout(dxout_hbm)


def _scatter_stage(pm, dxw, dst3):
    zero = jnp.zeros((N, D), jnp.float32)
    mesh = plsc.VectorSubcoreMesh(core_axis_name="c", subcore_axis_name="s")
    pm_sum, dxw_sum = pl.kernel(
        _sc_scatter_body,
        out_type=[jax.ShapeDtypeStruct((N, D), jnp.float32),
                  jax.ShapeDtypeStruct((N, D), jnp.float32)],
        mesh=mesh,
        scratch_types=[
            pltpu.VMEM_SHARED((N, D), jnp.float32),
            pltpu.VMEM((IDXROWS, CS), jnp.int32),
            pltpu.VMEM((CS, D), jnp.float32),
        ],
    )(pm, dxw, dst3, zero)
    return pm_sum, dxw_sum


def _node_body(h_ref, xp_ref, pms_ref, dxs_ref,
               wu1a_ref, wu1b_ref, bu1_ref, wu2_ref, bu2_ref,
               gamma_ref, beta_ref,
               hn_ref, xn_ref):
    h = h_ref[...]
    pms = pms_ref[...]
    f32 = jnp.float32
    u = _silu(jnp.dot(h, wu1a_ref[...], preferred_element_type=f32)
              + jnp.dot(pms, wu1b_ref[...], preferred_element_type=f32)
              + bu1_ref[...])
    upd = jnp.dot(u, wu2_ref[...], preferred_element_type=f32) + bu2_ref[...]
    pre = h + upd
    mu = jnp.mean(pre, axis=1, keepdims=True)
    cen = pre - mu
    var = jnp.mean(cen * cen, axis=1, keepdims=True)
    hn_ref[...] = cen * jax.lax.rsqrt(var + 1e-5) * gamma_ref[...] + beta_ref[...]
    xn_ref[...] = xp_ref[...] + dxs_ref[:, :XP]


def _node_stage(h, xp, pm_sum, dxw_sum, Wu1, bu1, Wu2, bu2, gamma, beta,
                *, interpret=False):
    n = h.shape[0]
    bn = min(BN, n)
    grid = (pl.cdiv(n, bn),)
    nb = lambda w: pl.BlockSpec((bn, w), lambda i: (i, 0))
    full = lambda a: pl.BlockSpec(a.shape, lambda i: (0,) * a.ndim)
    r2 = lambda v: v.reshape(1, -1)
    args = (h, xp, pm_sum, dxw_sum,
            Wu1[:D], Wu1[D:], r2(bu1), Wu2, r2(bu2), r2(gamma), r2(beta))
    in_specs = [nb(D), nb(XP), nb(D), nb(D)] + [full(a) for a in args[4:]]
    hn, xn = pl.pallas_call(
        _node_body,
        grid=grid,
        in_specs=in_specs,
        out_specs=[nb(D), nb(XP)],
        out_shape=[jax.ShapeDtypeStruct((n, D), jnp.float32),
                   jax.ShapeDtypeStruct((n, XP), jnp.float32)],
        interpret=interpret,
    )(*args)
    return hn, xn


def _chunk_index_rows(idx):
    idx2 = idx.reshape(NCHUNK, CS)
    main = idx2[:NS * CPT].reshape(NS, CPT, CS)
    extra = jnp.zeros((NS, IDXROWS - CPT, CS), jnp.int32)
    extra = extra.at[:NEXTRA, 0].set(idx2[NS * CPT:])
    return jnp.concatenate([main, extra], axis=1)


def kernel(h, x, e_emb, Wm1, bm1, Wm2, bm2, Wx1, bx1, Wx2, bx2, Wp1, bp1,
           Wp2, bp2, Wu1, bu1, Wu2, bu2, gamma, beta, edge_index,
           *, interpret=False):
    n = h.shape[0]
    src = edge_index[0]
    dst = edge_index[1]
    xp = jnp.pad(x, ((0, 0), (0, XP - x.shape[1])))

    xi = jnp.take(xp, dst, axis=0)
    xj = jnp.take(xp, src, axis=0)
    if interpret:
        hi = jnp.take(h, dst, axis=0)
        hj = jnp.take(h, src, axis=0)
    else:
        dst3 = _chunk_index_rows(dst)
        src3 = _chunk_index_rows(src)
        hi, hj = _gather_stage(h, dst3, src3)

    pm, dxw = _edge_stage(hi, hj, e_emb, xi, xj, Wm1, bm1, Wm2, bm2,
                          Wx1, bx1, Wx2, bx2, Wp1, bp1, Wp2, bp2,
                          interpret=interpret)

    if interpret:
        pm_sum = jax.ops.segment_sum(pm, dst, num_segments=n)
        dxw_sum = jax.ops.segment_sum(dxw, dst, num_segments=n)
    else:
        pm_sum, dxw_sum = _scatter_stage(pm, dxw, dst3)

    hn, xn = _node_stage(h, xp, pm_sum, dxw_sum, Wu1, bu1, Wu2, bu2,
                         gamma, beta, interpret=interpret)
    return hn, xn[:, :x.shape[1]]
